# Initial kernel scaffold; baseline (speedup 1.0000x reference)
import jax
import jax.numpy as jnp
from jax.experimental import pallas as pl


def kernel(hidden_states, mask_time_indices, codevectors, W, b):
    raise NotImplementedError("write your pallas kernel here")


# speedup vs baseline: 2.7217x; 2.7217x over previous
import functools

import jax
import jax.numpy as jnp
from jax import lax
from jax.experimental import pallas as pl
from jax.experimental.pallas import tpu as pltpu

G = 2
V = 320
D = 512
K = 128
TILE = 512


def _body(x_ref, maskf_ref, w_ref, b_ref, cb_ref, out_ref, ppl_ref,
          counts_ref, msum_ref, *, n_steps):
    i = pl.program_id(0)

    @pl.when(i == 0)
    def _init():
        counts_ref[...] = jnp.zeros_like(counts_ref)
        msum_ref[0, 0] = 0.0

    x = x_ref[...]
    maskf = maskf_ref[...]
    msum_ref[0, 0] += jnp.sum(maskf)

    for g in range(G):
        w_g = w_ref[g]
        logits = lax.dot_general(
            x, w_g, (((1,), (1,)), ((), ())),
            preferred_element_type=jnp.float32)
        logits = logits + b_ref[g][None, :]
        mx = jnp.max(logits, axis=-1, keepdims=True)
        iota = lax.broadcasted_iota(jnp.int32, logits.shape, 1)
        cand = jnp.where(logits == mx, iota, V)
        idx = jnp.min(cand, axis=-1, keepdims=True)
        oh = (iota == idx).astype(jnp.float32)
        counts_ref[g, :] += jnp.sum(oh * maskf, axis=0)
        out_ref[:, g * K:(g + 1) * K] = lax.dot_general(
            oh, cb_ref[g], (((1,), (0,)), ((), ())),
            preferred_element_type=jnp.float32)

    @pl.when(i == n_steps - 1)
    def _finalize():
        denom = jnp.maximum(msum_ref[0, 0], 1.0)
        avg = counts_ref[...] / denom
        plogp = avg * jnp.log(avg + 1e-07)
        neg = -jnp.sum(plogp, axis=1, keepdims=True)
        ppl_ref[...] = jnp.sum(jnp.exp(neg), axis=0, keepdims=True)


def kernel(hidden_states, mask_time_indices, codevectors, W, b):
    B, L, Dd = hidden_states.shape
    N = B * L
    n_steps = N // TILE
    x = hidden_states.reshape(N, Dd)
    maskf = mask_time_indices.reshape(N, 1).astype(jnp.float32)
    w3 = W.reshape(G, V, Dd)
    b2 = b.reshape(G, V)
    cb = codevectors.reshape(G, V, K)

    out, ppl = pl.pallas_call(
        functools.partial(_body, n_steps=n_steps),
        grid=(n_steps,),
        in_specs=[
            pl.BlockSpec((TILE, Dd), lambda i: (i, 0)),
            pl.BlockSpec((TILE, 1), lambda i: (i, 0)),
            pl.BlockSpec((G, V, Dd), lambda i: (0, 0, 0)),
            pl.BlockSpec((G, V), lambda i: (0, 0)),
            pl.BlockSpec((G, V, K), lambda i: (0, 0, 0)),
        ],
        out_specs=[
            pl.BlockSpec((TILE, G * K), lambda i: (i, 0)),
            pl.BlockSpec((1, 1), lambda i: (0, 0)),
        ],
        out_shape=[
            jax.ShapeDtypeStruct((N, G * K), jnp.float32),
            jax.ShapeDtypeStruct((1, 1), jnp.float32),
        ],
        scratch_shapes=[
            pltpu.VMEM((G, V), jnp.float32),
            pltpu.SMEM((1, 1), jnp.float32),
        ],
    )(x, maskf, w3, b2, cb)

    return out.reshape(B, L, G * K), ppl.reshape(())


# speedup vs baseline: 2.9740x; 1.0927x over previous
import functools

import jax
import jax.numpy as jnp
from jax import lax
from jax.experimental import pallas as pl
from jax.experimental.pallas import tpu as pltpu

G = 2
V = 320
D = 512
K = 128
TILE = 1024


def _body(x_ref, mrow_ref, w_ref, b_ref, cb_ref, out_ref, ppl_ref,
          counts_ref, *, n_steps):
    i = pl.program_id(0)

    @pl.when(i == 0)
    def _init():
        counts_ref[...] = jnp.zeros_like(counts_ref)

    x = x_ref[...]
    mrow = mrow_ref[0]

    for g in range(G):
        w_g = w_ref[g]
        logits = lax.dot_general(
            x, w_g, (((1,), (1,)), ((), ())),
            preferred_element_type=jnp.float32)
        logits = logits + b_ref[g][None, :]
        idx = jnp.argmax(logits, axis=-1)[:, None]
        iota = lax.broadcasted_iota(jnp.int32, logits.shape, 1)
        oh = (iota == idx).astype(jnp.float32)
        counts_ref[g:g + 1, :] += lax.dot_general(
            mrow, oh, (((1,), (0,)), ((), ())),
            preferred_element_type=jnp.float32)
        out_ref[:, g * K:(g + 1) * K] = lax.dot_general(
            oh, cb_ref[g], (((1,), (0,)), ((), ())),
            preferred_element_type=jnp.float32)

    @pl.when(i == n_steps - 1)
    def _finalize():
        counts = counts_ref[...]
        denom = jnp.maximum(jnp.sum(counts) * (1.0 / G), 1.0)
        avg = counts / denom
        plogp = avg * jnp.log(avg + 1e-07)
        neg = -jnp.sum(plogp, axis=1, keepdims=True)
        ppl_ref[...] = jnp.sum(jnp.exp(neg), axis=0, keepdims=True)


def kernel(hidden_states, mask_time_indices, codevectors, W, b):
    B, L, Dd = hidden_states.shape
    N = B * L
    n_steps = N // TILE
    x = hidden_states.reshape(N, Dd)
    mrow = mask_time_indices.reshape(n_steps, 1, TILE).astype(jnp.float32)
    w3 = W.reshape(G, V, Dd)
    b2 = b.reshape(G, V)
    cb = codevectors.reshape(G, V, K)

    out, ppl = pl.pallas_call(
        functools.partial(_body, n_steps=n_steps),
        grid=(n_steps,),
        in_specs=[
            pl.BlockSpec((TILE, Dd), lambda i: (i, 0)),
            pl.BlockSpec((1, 1, TILE), lambda i: (i, 0, 0)),
            pl.BlockSpec((G, V, Dd), lambda i: (0, 0, 0)),
            pl.BlockSpec((G, V), lambda i: (0, 0)),
            pl.BlockSpec((G, V, K), lambda i: (0, 0, 0)),
        ],
        out_specs=[
            pl.BlockSpec((TILE, G * K), lambda i: (i, 0)),
            pl.BlockSpec((1, 1), lambda i: (0, 0)),
        ],
        out_shape=[
            jax.ShapeDtypeStruct((N, G * K), jnp.float32),
            jax.ShapeDtypeStruct((1, 1), jnp.float32),
        ],
        scratch_shapes=[
            pltpu.VMEM((G, V), jnp.float32),
        ],
    )(x, mrow, w3, b2, cb)

    return out.reshape(B, L, G * K), ppl.reshape(())


# speedup vs baseline: 4.3056x; 1.4477x over previous
import functools

import jax
import jax.numpy as jnp
from jax import lax
from jax.experimental import pallas as pl
from jax.experimental.pallas import tpu as pltpu

G = 2
V = 320
D = 512
K = 128
TILE = 1024


def _body(x_ref, mrow_ref, w_ref, b_ref, cb_ref, out_ref, ppl_ref,
          counts_ref, idx_ref, *, n_steps):
    i = pl.program_id(0)

    @pl.when(i == 0)
    def _init():
        counts_ref[...] = jnp.zeros_like(counts_ref)

    idx_prev = [idx_ref[g][...] for g in range(G)]

    x = x_ref[...]
    iota = lax.broadcasted_iota(jnp.int32, (TILE, V), 1)
    for g in range(G):
        logits = lax.dot_general(
            x, w_ref[g], (((1,), (1,)), ((), ())),
            preferred_element_type=jnp.float32)
        logits = logits + b_ref[g][None, :]
        mx = jnp.max(logits, axis=-1, keepdims=True)
        cand = jnp.where(logits == mx, iota, V)
        idx_ref[g] = jnp.min(cand, axis=-1, keepdims=True)

    mrow = mrow_ref[0]
    live = jnp.where(i > 0, 1.0, 0.0)
    for g in range(G):
        oh = (iota == idx_prev[g]).astype(jnp.float32)
        counts_ref[g:g + 1, :] += live * lax.dot_general(
            mrow, oh, (((1,), (0,)), ((), ())),
            preferred_element_type=jnp.float32)
        out_ref[:, g * K:(g + 1) * K] = lax.dot_general(
            oh, cb_ref[g], (((1,), (0,)), ((), ())),
            preferred_element_type=jnp.float32)

    @pl.when(i == n_steps)
    def _finalize():
        counts = counts_ref[...]
        denom = jnp.maximum(jnp.sum(counts) * (1.0 / G), 1.0)
        avg = counts / denom
        plogp = avg * jnp.log(avg + 1e-07)
        neg = -jnp.sum(plogp, axis=1, keepdims=True)
        ppl_ref[...] = jnp.sum(jnp.exp(neg), axis=0, keepdims=True)


def kernel(hidden_states, mask_time_indices, codevectors, W, b):
    B, L, Dd = hidden_states.shape
    N = B * L
    n_steps = N // TILE
    x = hidden_states.reshape(N, Dd)
    mrow = mask_time_indices.reshape(n_steps, 1, TILE).astype(jnp.float32)
    w3 = W.reshape(G, V, Dd)
    b2 = b.reshape(G, V)
    cb = codevectors.reshape(G, V, K)

    out, ppl = pl.pallas_call(
        functools.partial(_body, n_steps=n_steps),
        grid=(n_steps + 1,),
        in_specs=[
            pl.BlockSpec((TILE, Dd), lambda i: (jnp.minimum(i, n_steps - 1), 0)),
            pl.BlockSpec((1, 1, TILE), lambda i: (jnp.maximum(i - 1, 0), 0, 0)),
            pl.BlockSpec((G, V, Dd), lambda i: (0, 0, 0)),
            pl.BlockSpec((G, V), lambda i: (0, 0)),
            pl.BlockSpec((G, V, K), lambda i: (0, 0, 0)),
        ],
        out_specs=[
            pl.BlockSpec((TILE, G * K), lambda i: (jnp.maximum(i - 1, 0), 0)),
            pl.BlockSpec((1, 1), lambda i: (0, 0)),
        ],
        out_shape=[
            jax.ShapeDtypeStruct((N, G * K), jnp.float32),
            jax.ShapeDtypeStruct((1, 1), jnp.float32),
        ],
        scratch_shapes=[
            pltpu.VMEM((G, V), jnp.float32),
            pltpu.VMEM((G, TILE, 1), jnp.int32),
        ],
    )(x, mrow, w3, b2, cb)

    return out.reshape(B, L, G * K), ppl.reshape(())


# speedup vs baseline: 4.5196x; 1.0497x over previous
import functools

import jax
import jax.numpy as jnp
from jax import lax
from jax.experimental import pallas as pl
from jax.experimental.pallas import tpu as pltpu

G = 2
V = 320
D = 512
K = 128
TILE = 1024


def _body(x_ref, mrow_ref, w_ref, b_ref, cb_ref, out_ref, ppl_ref,
          counts_ref, idx_ref, *, n_steps):
    i = pl.program_id(0)

    @pl.when(i == 0)
    def _init():
        counts_ref[...] = jnp.zeros_like(counts_ref)

    idx_prev = [idx_ref[g][...] for g in range(G)]

    x = x_ref[...]
    iota = lax.broadcasted_iota(jnp.int32, (TILE, V), 1).astype(jnp.float32)
    for g in range(G):
        logits = lax.dot_general(
            x, w_ref[g], (((1,), (1,)), ((), ())),
            preferred_element_type=jnp.float32)
        logits = logits + b_ref[g][None, :]
        mx = jnp.max(logits, axis=-1, keepdims=True)
        cand = jnp.where(logits == mx, iota, float(V))
        idx_ref[g] = jnp.min(cand, axis=-1, keepdims=True)

    mrow = mrow_ref[0]
    live = jnp.where(i > 0, 1.0, 0.0)
    for g in range(G):
        oh = (iota == idx_prev[g]).astype(jnp.float32)
        counts_ref[g:g + 1, :] += live * lax.dot_general(
            mrow, oh, (((1,), (0,)), ((), ())),
            preferred_element_type=jnp.float32)
        out_ref[:, g * K:(g + 1) * K] = lax.dot_general(
            oh, cb_ref[g], (((1,), (0,)), ((), ())),
            preferred_element_type=jnp.float32)

    @pl.when(i == n_steps)
    def _finalize():
        counts = counts_ref[...]
        denom = jnp.maximum(jnp.sum(counts) * (1.0 / G), 1.0)
        avg = counts / denom
        plogp = avg * jnp.log(avg + 1e-07)
        neg = -jnp.sum(plogp, axis=1, keepdims=True)
        ppl_ref[...] = jnp.sum(jnp.exp(neg), axis=0, keepdims=True)


def kernel(hidden_states, mask_time_indices, codevectors, W, b):
    B, L, Dd = hidden_states.shape
    N = B * L
    n_steps = N // TILE
    x = hidden_states.reshape(N, Dd)
    mrow = mask_time_indices.reshape(n_steps, 1, TILE).astype(jnp.float32)
    w3 = W.reshape(G, V, Dd)
    b2 = b.reshape(G, V)
    cb = codevectors.reshape(G, V, K)

    out, ppl = pl.pallas_call(
        functools.partial(_body, n_steps=n_steps),
        grid=(n_steps + 1,),
        in_specs=[
            pl.BlockSpec((TILE, Dd), lambda i: (jnp.minimum(i, n_steps - 1), 0)),
            pl.BlockSpec((1, 1, TILE), lambda i: (jnp.maximum(i - 1, 0), 0, 0)),
            pl.BlockSpec((G, V, Dd), lambda i: (0, 0, 0)),
            pl.BlockSpec((G, V), lambda i: (0, 0)),
            pl.BlockSpec((G, V, K), lambda i: (0, 0, 0)),
        ],
        out_specs=[
            pl.BlockSpec((TILE, G * K), lambda i: (jnp.maximum(i - 1, 0), 0)),
            pl.BlockSpec((1, 1), lambda i: (0, 0)),
        ],
        out_shape=[
            jax.ShapeDtypeStruct((N, G * K), jnp.float32),
            jax.ShapeDtypeStruct((1, 1), jnp.float32),
        ],
        scratch_shapes=[
            pltpu.VMEM((G, V), jnp.float32),
            pltpu.VMEM((G, TILE, 1), jnp.float32),
        ],
    )(x, mrow, w3, b2, cb)

    return out.reshape(B, L, G * K), ppl.reshape(())
